# baseline (speedup 1.0000x reference)
import functools

import jax
import jax.numpy as jnp
from jax import lax
from jax.experimental import pallas as pl
from jax.experimental.pallas import tpu as pltpu
from jax.experimental.pallas import tpu_sc as plsc

BUCKET_NUMBER = 1000
MIN_VALUE = 0.0
MAX_VALUE = 1.0
STEP = (MAX_VALUE - MIN_VALUE) / BUCKET_NUMBER
EMBED_DIM = 64

LANES = 16
IDX_BLK = 128
CHUNK = 256
NBUF = 4


def _make_kernel(B, D):
    info = plsc.get_sparse_core_info()
    NC, NS = info.num_cores, info.num_subcores
    NW = NC * NS
    assert B % (NW * CHUNK * NBUF) == 0
    per_w = B // NW
    n_iters = per_w // (CHUNK * NBUF)
    n_blk = CHUNK // IDX_BLK

    mesh = plsc.VectorSubcoreMesh(core_axis_name="c", subcore_axis_name="s")

    @functools.partial(
        pl.kernel,
        out_type=jax.ShapeDtypeStruct((B, D), jnp.float32),
        mesh=mesh,
        scratch_types=[
            pltpu.VMEM((NBUF, CHUNK), jnp.float32),
            pltpu.VMEM((NBUF, n_blk, IDX_BLK), jnp.int32),
            pltpu.VMEM((NBUF, CHUNK, D), jnp.float32),
            pltpu.SemaphoreType.DMA,
        ]
        + [pltpu.SemaphoreType.DMA for _ in range(NBUF)],
        compiler_params=pltpu.CompilerParams(use_tc_tiling_on_sc=False),
    )
    def k(x_hbm, table_hbm, out_hbm, x_v, idx_v, rows_v, gsem, *ssems):
        wid = lax.axis_index("s") * NC + lax.axis_index("c")
        base = wid * per_w

        def iter_body(t, carry):
            handles = []
            for k_ in range(NBUF):
                cbase = base + (t * NBUF + k_) * CHUNK
                pltpu.sync_copy(x_hbm.at[pl.ds(cbase, CHUNK)], x_v.at[k_])
                for i in range(CHUNK // LANES):
                    v = x_v[k_, pl.ds(i * LANES, LANES)]
                    t_ = (v - MIN_VALUE) / STEP
                    idx = t_.astype(jnp.int32)
                    idx = jnp.minimum(jnp.maximum(idx, 0), BUCKET_NUMBER - 1)
                    j, o = divmod(i * LANES, IDX_BLK)
                    idx_v[k_, j, pl.ds(o, LANES)] = idx

                @pl.when(t > 0)
                def _(k_=k_, cbase=cbase):
                    pltpu.make_async_copy(
                        rows_v.at[k_],
                        out_hbm.at[pl.ds(cbase - NBUF * CHUNK, CHUNK)],
                        ssems[k_],
                    ).wait()

                handles.append([
                    pltpu.async_copy(
                        table_hbm.at[idx_v.at[k_, j]],
                        rows_v.at[k_, pl.ds(j * IDX_BLK, IDX_BLK)],
                        gsem,
                    )
                    for j in range(n_blk)
                ])
            for k_ in range(NBUF):
                cbase = base + (t * NBUF + k_) * CHUNK
                for h in handles[k_]:
                    h.wait()
                pltpu.async_copy(
                    rows_v.at[k_],
                    out_hbm.at[pl.ds(cbase, CHUNK)],
                    ssems[k_],
                )
            return carry

        lax.fori_loop(0, n_iters, iter_body, 0)

        for k_ in range(NBUF):
            cbase = base + ((n_iters - 1) * NBUF + k_) * CHUNK
            pltpu.make_async_copy(
                rows_v.at[k_],
                out_hbm.at[pl.ds(cbase, CHUNK)],
                ssems[k_],
            ).wait()

    return k


def kernel(x, table):
    if x.ndim == 2 and x.shape[1] == 1:
        x = jnp.squeeze(x, axis=-1)
    shape = x.shape
    B = x.size
    xf = x.reshape(B)
    out = _make_kernel(B, table.shape[1])(xf, table)
    return out.reshape(*shape, table.shape[1])
